# baseline (speedup 1.0000x reference)
import functools

import jax
import jax.numpy as jnp
from jax import lax
from jax.experimental import pallas as pl
from jax.experimental.pallas import tpu as pltpu
from jax.experimental.pallas import tpu_sc as plsc

CHUNK = 128
NBUF = 4


@functools.cache
def _make_lookup(B, L, V, D):
    info = plsc.get_sparse_core_info()
    nc = info.num_cores
    nw = nc * info.num_subcores
    n_units = B * L // CHUNK
    u_per_w = n_units // nw
    bc_per_l = B // CHUNK
    assert u_per_w % NBUF == 0
    mesh = plsc.VectorSubcoreMesh(core_axis_name="c", subcore_axis_name="s")

    @functools.partial(
        pl.kernel,
        mesh=mesh,
        out_type=jax.ShapeDtypeStruct((L, D, B), jnp.float32),
        compiler_params=pltpu.CompilerParams(
            use_tc_tiling_on_sc=True, needs_layout_passes=False
        ),
        scratch_types=[
            pltpu.VMEM((u_per_w, CHUNK), jnp.int32),
            pltpu.VMEM((NBUF, CHUNK, 2 * D), jnp.float32),
            pltpu.VMEM((NBUF, D, CHUNK), jnp.float32),
            [pltpu.SemaphoreType.DMA] * NBUF,
            [pltpu.SemaphoreType.DMA] * NBUF,
        ],
    )
    def lookup(idx_hbm, table_hbm, out_hbm, idx_v, pbuf, obuf, gsems, osems):
        wid = lax.axis_index("s") * nc + lax.axis_index("c")
        ubase = wid * u_per_w
        pltpu.sync_copy(idx_hbm.at[pl.ds(ubase, u_per_w)], idx_v)
        iota = lax.iota(jnp.int32, 16)

        def fire_gather(t, b):
            for g in range(CHUNK // 16):
                iv = idx_v[t, pl.ds(16 * g, 16)] >> 1
                pltpu.async_copy(
                    table_hbm.at[iv], pbuf.at[b, pl.ds(16 * g, 16)], gsems[b]
                )

        def drain_gather(b):
            for g in range(CHUNK // 16):
                pltpu.make_async_copy(
                    table_hbm.at[iota], pbuf.at[b, pl.ds(16 * g, 16)],
                    gsems[b],
                ).wait()

        def select(t, b):
            for g in range(CHUNK // 16):
                jv = iota + 16 * g
                sv = (idx_v[t, pl.ds(16 * g, 16)] & 1) << 6

                @plsc.parallel_loop(0, D, unroll=8)
                def _(d):
                    vals = plsc.load_gather(pbuf.at[b], [jv, sv + d])
                    obuf.at[b][d, pl.ds(16 * g, 16)] = vals

        def out_slice(t):
            u = ubase + t
            l = u // bc_per_l
            bc = u % bc_per_l
            return out_hbm.at[l, :, pl.ds(bc * CHUNK, CHUNK)]

        def fire_out(t, b):
            pltpu.async_copy(obuf.at[b], out_slice(t), osems[b])

        def wait_out(t, b):
            pltpu.make_async_copy(obuf.at[b], out_slice(t), osems[b]).wait()

        for b in range(NBUF):
            fire_gather(b, b)

        def body(i, carry):
            for b in range(NBUF):
                t = NBUF * i + b
                drain_gather(b)
                pl.when(t >= NBUF)(lambda: wait_out(t - NBUF, b))
                select(t, b)
                fire_out(t, b)
                pl.when(t + NBUF < u_per_w)(lambda: fire_gather(t + NBUF, b))
            return carry

        lax.fori_loop(0, u_per_w // NBUF, body, 0)
        for b in range(NBUF):
            wait_out(u_per_w - NBUF + b, b)

    return lookup


def kernel(x, table):
    B, L = x.shape
    V, D = table.shape
    xt = x.astype(jnp.int32).T.reshape(L * B // CHUNK, CHUNK)
    tpack = table.reshape(V // 2, 2 * D)
    out = _make_lookup(B, L, V, D)(xt, tpack)
    return out.transpose(2, 0, 1)


# speedup vs baseline: 1.0001x; 1.0001x over previous
import functools

import jax
import jax.numpy as jnp
from jax import lax
from jax.experimental import pallas as pl
from jax.experimental.pallas import tpu as pltpu
from jax.experimental.pallas import tpu_sc as plsc

CHUNK = 128
NBUF = 4


@functools.cache
def _make_lookup(B, L, V, D):
    info = plsc.get_sparse_core_info()
    nc = info.num_cores
    nw = nc * info.num_subcores
    n_units = B * L // CHUNK
    u_per_w = n_units // nw
    bc_per_l = B // CHUNK
    assert u_per_w % NBUF == 0
    mesh = plsc.VectorSubcoreMesh(core_axis_name="c", subcore_axis_name="s")

    @functools.partial(
        pl.kernel,
        mesh=mesh,
        out_type=jax.ShapeDtypeStruct((L, D, B), jnp.float32),
        compiler_params=pltpu.CompilerParams(
            use_tc_tiling_on_sc=True, needs_layout_passes=False
        ),
        scratch_types=[
            pltpu.VMEM((u_per_w, CHUNK), jnp.int32),
            pltpu.VMEM((NBUF, CHUNK, 2 * D), jnp.float32),
            pltpu.VMEM((NBUF, D, CHUNK), jnp.float32),
            [pltpu.SemaphoreType.DMA] * NBUF,
            [pltpu.SemaphoreType.DMA] * NBUF,
        ],
    )
    def lookup(idx_hbm, table_hbm, out_hbm, idx_v, pbuf, obuf, gsems, osems):
        wid = lax.axis_index("s") * nc + lax.axis_index("c")
        ubase = wid * u_per_w
        pltpu.sync_copy(idx_hbm.at[pl.ds(ubase, u_per_w)], idx_v)
        iota = lax.iota(jnp.int32, 16)

        def fire_gather(t, b):
            for g in range(CHUNK // 16):
                iv = idx_v[t, pl.ds(16 * g, 16)] >> 1
                pltpu.async_copy(
                    table_hbm.at[iv], pbuf.at[b, pl.ds(16 * g, 16)], gsems[b]
                )

        def drain_gather(b):
            for g in range(CHUNK // 16):
                pltpu.make_async_copy(
                    table_hbm.at[iota], pbuf.at[b, pl.ds(16 * g, 16)],
                    gsems[b],
                ).wait()

        def select(t, b):
            def sel_g(g, carry):
                jv = iota + 16 * g
                sv = (idx_v[t, pl.ds(16 * g, 16)] & 1) << 6

                @plsc.parallel_loop(0, D, unroll=4)
                def _(d):
                    vals = plsc.load_gather(pbuf.at[b], [jv, sv + d])
                    obuf.at[b][d, pl.ds(16 * g, 16)] = vals

                return carry

            lax.fori_loop(0, CHUNK // 16, sel_g, 0)

        def out_slice(t):
            u = ubase + t
            l = u // bc_per_l
            bc = u % bc_per_l
            return out_hbm.at[l, :, pl.ds(bc * CHUNK, CHUNK)]

        def fire_out(t, b):
            pltpu.async_copy(obuf.at[b], out_slice(t), osems[b])

        def wait_out(t, b):
            pltpu.make_async_copy(obuf.at[b], out_slice(t), osems[b]).wait()

        for b in range(NBUF):
            fire_gather(b, b)

        def body(i, carry):
            for b in range(NBUF):
                t = NBUF * i + b
                drain_gather(b)
                pl.when(t >= NBUF)(lambda: wait_out(t - NBUF, b))
                select(t, b)
                fire_out(t, b)
                pl.when(t + NBUF < u_per_w)(lambda: fire_gather(t + NBUF, b))
            return carry

        lax.fori_loop(0, u_per_w // NBUF, body, 0)
        for b in range(NBUF):
            wait_out(u_per_w - NBUF + b, b)

    return lookup


def kernel(x, table):
    B, L = x.shape
    V, D = table.shape
    xt = x.astype(jnp.int32).T.reshape(L * B // CHUNK, CHUNK)
    tpack = table.reshape(V // 2, 2 * D)
    out = _make_lookup(B, L, V, D)(xt, tpack)
    return out.transpose(2, 0, 1)


# speedup vs baseline: 1.0440x; 1.0439x over previous
import functools

import jax
import jax.numpy as jnp
from jax import lax
from jax.experimental import pallas as pl
from jax.experimental.pallas import tpu as pltpu
from jax.experimental.pallas import tpu_sc as plsc

CHUNK = 128
K = 2
GROUP = K * CHUNK
NBUF = 4


@functools.cache
def _make_lookup(N, D):
    info = plsc.get_sparse_core_info()
    nw = info.num_cores * info.num_subcores
    b_per_w = N // nw
    n_chunks = b_per_w // CHUNK
    n_groups = n_chunks // K
    n_main = n_groups - NBUF
    assert n_main % NBUF == 0
    mesh = plsc.VectorSubcoreMesh(core_axis_name="c", subcore_axis_name="s")

    @functools.partial(
        pl.kernel,
        mesh=mesh,
        out_type=jax.ShapeDtypeStruct((N, D), jnp.float32),
        compiler_params=pltpu.CompilerParams(use_tc_tiling_on_sc=False),
        scratch_types=[
            pltpu.VMEM((n_chunks, CHUNK), jnp.int32),
            pltpu.VMEM((NBUF, GROUP, D), jnp.float32),
            [pltpu.SemaphoreType.DMA] * NBUF,
            [pltpu.SemaphoreType.DMA] * NBUF,
        ],
    )
    def lookup(idx_hbm, table_hbm, out_hbm, idx_v, bufs, gsems, osems):
        wid = lax.axis_index("s") * info.num_cores + lax.axis_index("c")
        base = wid * b_per_w
        pltpu.sync_copy(idx_hbm.at[pl.ds(wid * n_chunks, n_chunks)], idx_v)

        def fire_gathers(g, b):
            for j in range(K):
                pltpu.async_copy(
                    table_hbm.at[idx_v.at[g * K + j]],
                    bufs.at[b, pl.ds(j * CHUNK, CHUNK)],
                    gsems[b],
                )

        def drain_gathers(g, b):
            for j in range(K):
                pltpu.make_async_copy(
                    table_hbm.at[idx_v.at[g * K + j]],
                    bufs.at[b, pl.ds(j * CHUNK, CHUNK)],
                    gsems[b],
                ).wait()

        def fire_out(g, b):
            pltpu.async_copy(
                bufs.at[b], out_hbm.at[pl.ds(base + g * GROUP, GROUP)],
                osems[b],
            )

        def wait_out(g, b):
            pltpu.make_async_copy(
                bufs.at[b], out_hbm.at[pl.ds(base + g * GROUP, GROUP)],
                osems[b],
            ).wait()

        for b in range(NBUF):
            fire_gathers(b, b)

        def body(i, carry):
            for b in range(NBUF):
                t = i * NBUF + b
                drain_gathers(t, b)
                fire_out(t, b)
                wait_out(t, b)
                fire_gathers(t + NBUF, b)
            return carry

        lax.fori_loop(0, n_main // NBUF, body, 0)

        for b in range(NBUF):
            t = n_main + b
            drain_gathers(t, b)
            fire_out(t, b)
        for b in range(NBUF):
            wait_out(n_main + b, b)

    return lookup


def kernel(x, table):
    B, L = x.shape
    D = table.shape[1]
    idx = x.reshape(-1, CHUNK).astype(jnp.int32)
    out = _make_lookup(B * L, D)(idx, table)
    return out.reshape(B, L, D)
